# baseline (speedup 1.0000x reference)
import jax
import jax.numpy as jnp
from jax.experimental import pallas as pl
from jax.experimental.pallas import tpu as pltpu

NB = 4096
NC = 8192
DH = 256
BLK = 4096
NBLK = 1024
BW = 256
SUB = BLK // BW
XS = BLK // NBLK

N_X0 = NC // NBLK
N_BODY = NB // BW
N_MID = NC // BLK
GRID = N_X0 + N_BODY + N_MID


def _conv_kernel(notes_ref, weight_ref, w_ref, b_ref, out_ref, x0c_ref):
    i = pl.program_id(0)

    @pl.when(i < N_X0)
    def _x0_phase():
        blk = jnp.dot(notes_ref[...].astype(jnp.bfloat16),
                      w_ref[...].astype(jnp.bfloat16),
                      preferred_element_type=jnp.float32)
        x0c_ref[pl.ds(i * NBLK, NBLK), :] = blk.astype(jnp.bfloat16)
        r = i % XS
        out_ref[pl.ds(r * NBLK, NBLK), :] = blk

    @pl.when((i >= N_X0) & (i < N_X0 + N_BODY))
    def _body_phase():
        j = i - N_X0
        r = j % SUB
        rn = (j * BW) % NBLK
        x0b = jnp.dot(notes_ref[pl.ds(rn, BW), :].astype(jnp.bfloat16),
                      w_ref[...].astype(jnp.bfloat16),
                      preferred_element_type=jnp.float32)
        acc = x0b + jnp.dot(weight_ref[...].astype(jnp.bfloat16),
                            x0c_ref[...],
                            preferred_element_type=jnp.float32)
        out_ref[pl.ds(r * BW, BW), :] = jnp.maximum(acc + b_ref[...], 0.0)

    @pl.when(i >= N_X0 + N_BODY)
    def _fill_phase():
        out_ref[...] = jnp.broadcast_to(
            jnp.maximum(b_ref[...], 0.0), (BLK, DH))


def _notes_map(i):
    return (jnp.where(i < N_X0, i + NB // NBLK,
                      jnp.minimum((i - N_X0) * BW // NBLK,
                                  NB // NBLK - 1)), 0)


def _weight_map(i):
    return (jnp.clip(i - N_X0, 0, N_BODY - 1), 0)


def _out_map(i):
    return (jnp.where(
        i < N_X0, i // XS + (NB + NC) // BLK,
        jnp.where(i < N_X0 + N_BODY, (i - N_X0) // SUB,
                  NB // BLK + (i - N_X0 - N_BODY))), 0)


def kernel(notes, weight, w, b):
    b2 = b.reshape(1, DH)
    return pl.pallas_call(
        _conv_kernel,
        grid=(GRID,),
        in_specs=[
            pl.BlockSpec((NBLK, DH), _notes_map),
            pl.BlockSpec((BW, NC), _weight_map),
            pl.BlockSpec((DH, DH), lambda i: (0, 0)),
            pl.BlockSpec((1, DH), lambda i: (0, 0)),
        ],
        out_specs=pl.BlockSpec((BLK, DH), _out_map),
        out_shape=jax.ShapeDtypeStruct((NB + 2 * NC, DH), jnp.float32),
        scratch_shapes=[pltpu.VMEM((NC, DH), jnp.bfloat16)],
    )(notes, weight, w, b2)


# speedup vs baseline: 1.0719x; 1.0719x over previous
import jax
import jax.numpy as jnp
from jax.experimental import pallas as pl
from jax.experimental.pallas import tpu as pltpu

NB = 4096
NC = 8192
DH = 256
BLK = 4096
BW = 256
SUB = BLK // BW

N_X0 = NC // BLK
N_BODY = NB // BW
N_MID = NC // BLK
GRID = N_X0 + N_BODY + N_MID


def _conv_kernel(notes_ref, weight_ref, w_ref, b_ref, out_ref, x0c_ref):
    i = pl.program_id(0)

    @pl.when(i < N_X0)
    def _x0_phase():
        blk = jnp.dot(notes_ref[...].astype(jnp.bfloat16),
                      w_ref[...].astype(jnp.bfloat16),
                      preferred_element_type=jnp.float32)
        x0c_ref[pl.ds(i * BLK, BLK), :] = blk.astype(jnp.bfloat16)
        out_ref[...] = blk

    @pl.when((i >= N_X0) & (i < N_X0 + N_BODY))
    def _body_phase():
        r = (i - N_X0) % SUB
        x0b = jnp.dot(notes_ref[pl.ds(r * BW, BW), :].astype(jnp.bfloat16),
                      w_ref[...].astype(jnp.bfloat16),
                      preferred_element_type=jnp.float32)
        acc = x0b + jnp.dot(weight_ref[...].astype(jnp.bfloat16),
                            x0c_ref[...],
                            preferred_element_type=jnp.float32)
        out_ref[pl.ds(r * BW, BW), :] = jnp.maximum(acc + b_ref[...], 0.0)

    @pl.when(i >= N_X0 + N_BODY)
    def _fill_phase():
        out_ref[...] = jnp.broadcast_to(
            jnp.maximum(b_ref[...], 0.0), (BLK, DH))


def _notes_map(i):
    return (jnp.where(i < N_X0, i + NB // BLK,
                      jnp.minimum((i - N_X0) // SUB, NB // BLK - 1)), 0)


def _weight_map(i):
    return (jnp.clip(i - N_X0, 0, N_BODY - 1), 0)


def _out_map(i):
    return (jnp.where(
        i < N_X0, i + (NB + NC) // BLK,
        jnp.where(i < N_X0 + N_BODY, (i - N_X0) // SUB,
                  NB // BLK + (i - N_X0 - N_BODY))), 0)


def kernel(notes, weight, w, b):
    b2 = b.reshape(1, DH)
    return pl.pallas_call(
        _conv_kernel,
        grid=(GRID,),
        in_specs=[
            pl.BlockSpec((BLK, DH), _notes_map),
            pl.BlockSpec((BW, NC), _weight_map),
            pl.BlockSpec((DH, DH), lambda i: (0, 0)),
            pl.BlockSpec((1, DH), lambda i: (0, 0)),
        ],
        out_specs=pl.BlockSpec((BLK, DH), _out_map),
        out_shape=jax.ShapeDtypeStruct((NB + 2 * NC, DH), jnp.float32),
        scratch_shapes=[pltpu.VMEM((NC, DH), jnp.bfloat16)],
    )(notes, weight, w, b2)
